# Initial kernel scaffold; baseline (speedup 1.0000x reference)
import jax
import jax.numpy as jnp
from jax.experimental import pallas as pl


def kernel(map_ids, commander_ids, mutation_ids, ai_ids, map_table, commander_table, mutation_table, ai_table):
    raise NotImplementedError("write your pallas kernel here")


# speedup vs baseline: 11.0688x; 11.0688x over previous
import functools

import jax
import jax.numpy as jnp
from jax import lax
from jax.experimental import pallas as pl
from jax.experimental.pallas import tpu as pltpu
from jax.experimental.pallas import tpu_sc as plsc

NUM_CORES = 2
NUM_SUBCORES = 16
NUM_WORKERS = NUM_CORES * NUM_SUBCORES
BATCH = 16384
ROWS_PER_WORKER = BATCH // NUM_WORKERS
CHUNK = 64
NUM_CHUNKS = ROWS_PER_WORKER // CHUNK
MUT_LEN = 50
MAP_DIM = 64
CMD_DIM = 128
MUT_DIM = 96
AI_DIM = 32
OUT_DIM = MAP_DIM + 2 * CMD_DIM + MUT_DIM + AI_DIM
MUT_VECS = MUT_DIM // 16


def _sc_body(map_ids_h, cmd_even_h, cmd_odd_h, mut_ids_h, ai_ids_h,
             map_t_h, cmd_t_h, mut_t_h, ai_t_h, out_h,
             mut_tab_v, mut_ids_v, mut_out_v,
             map_idx_v, map_rows_v,
             cmde_idx_v, cmde_rows_v,
             cmdo_idx_v, cmdo_rows_v,
             ai_idx_v, ai_rows_v,
             sem):
  wid = lax.axis_index("s") * NUM_CORES + lax.axis_index("c")
  base = wid * ROWS_PER_WORKER

  pltpu.sync_copy(mut_t_h, mut_tab_v)

  @pl.loop(0, NUM_CHUNKS)
  def _(k):
    r = base + k * CHUNK

    pltpu.sync_copy(map_ids_h.at[pl.ds(r, CHUNK)], map_idx_v)
    pltpu.sync_copy(cmd_even_h.at[pl.ds(r, CHUNK)], cmde_idx_v)
    pltpu.sync_copy(cmd_odd_h.at[pl.ds(r, CHUNK)], cmdo_idx_v)
    pltpu.sync_copy(ai_ids_h.at[pl.ds(r, CHUNK)], ai_idx_v)
    pltpu.sync_copy(mut_ids_h.at[pl.ds(r * MUT_LEN, CHUNK * MUT_LEN)],
                    mut_ids_v.at[pl.ds(0, CHUNK * MUT_LEN)])

    cp_map = pltpu.async_copy(map_t_h.at[map_idx_v], map_rows_v, sem)
    cp_cmde = pltpu.async_copy(cmd_t_h.at[cmde_idx_v], cmde_rows_v, sem)
    cp_cmdo = pltpu.async_copy(cmd_t_h.at[cmdo_idx_v], cmdo_rows_v, sem)
    cp_ai = pltpu.async_copy(ai_t_h.at[ai_idx_v], ai_rows_v, sem)

    @pl.loop(0, CHUNK)
    def _(s):
      sbase = s * MUT_LEN
      idv = [mut_ids_v[pl.ds(sbase + 16 * t, 16)] for t in range(4)]
      accs = [jnp.zeros((16,), jnp.float32) for _ in range(MUT_VECS)]
      for j in range(MUT_LEN):
        off = idv[j // 16][j % 16] * MUT_DIM
        for g in range(MUT_VECS):
          accs[g] = accs[g] + mut_tab_v[pl.ds(off + g * 16, 16)]
      scale = jnp.float32(1.0 / MUT_LEN)
      for g in range(MUT_VECS):
        mut_out_v[s, pl.ds(g * 16, 16)] = accs[g] * scale

    cp_map.wait()
    cp_cmde.wait()
    cp_cmdo.wait()
    cp_ai.wait()

    pltpu.sync_copy(map_rows_v, out_h.at[pl.ds(r, CHUNK), pl.ds(0, MAP_DIM)])
    pltpu.sync_copy(cmde_rows_v,
                    out_h.at[pl.ds(r, CHUNK), pl.ds(MAP_DIM, CMD_DIM)])
    pltpu.sync_copy(cmdo_rows_v,
                    out_h.at[pl.ds(r, CHUNK), pl.ds(MAP_DIM + CMD_DIM,
                                                    CMD_DIM)])
    pltpu.sync_copy(mut_out_v,
                    out_h.at[pl.ds(r, CHUNK), pl.ds(MAP_DIM + 2 * CMD_DIM,
                                                    MUT_DIM)])
    pltpu.sync_copy(ai_rows_v,
                    out_h.at[pl.ds(r, CHUNK), pl.ds(OUT_DIM - AI_DIM,
                                                    AI_DIM)])


@jax.jit
def _embed(map_ids, cmd_even, cmd_odd, mut_ids_flat, ai_ids,
           map_table, cmd_table, mut_table_flat, ai_table):
  mesh = plsc.VectorSubcoreMesh(core_axis_name="c", subcore_axis_name="s",
                                num_cores=NUM_CORES,
                                num_subcores=NUM_SUBCORES)
  run = functools.partial(
      pl.kernel,
      out_type=jax.ShapeDtypeStruct((BATCH, OUT_DIM), jnp.float32),
      mesh=mesh,
      compiler_params=pltpu.CompilerParams(use_tc_tiling_on_sc=False),
      scratch_types=[
          pltpu.VMEM((1000 * MUT_DIM,), jnp.float32),
          pltpu.VMEM((CHUNK * MUT_LEN + 16,), jnp.int32),
          pltpu.VMEM((CHUNK, MUT_DIM), jnp.float32),
          pltpu.VMEM((CHUNK,), jnp.int32),
          pltpu.VMEM((CHUNK, MAP_DIM), jnp.float32),
          pltpu.VMEM((CHUNK,), jnp.int32),
          pltpu.VMEM((CHUNK, CMD_DIM), jnp.float32),
          pltpu.VMEM((CHUNK,), jnp.int32),
          pltpu.VMEM((CHUNK, CMD_DIM), jnp.float32),
          pltpu.VMEM((CHUNK,), jnp.int32),
          pltpu.VMEM((CHUNK, AI_DIM), jnp.float32),
          pltpu.SemaphoreType.DMA,
      ],
  )(_sc_body)
  return run(map_ids, cmd_even, cmd_odd, mut_ids_flat, ai_ids,
             map_table, cmd_table, mut_table_flat, ai_table)


def kernel(map_ids, commander_ids, mutation_ids, ai_ids,
           map_table, commander_table, mutation_table, ai_table):
  cmd_even = commander_ids[:, 0]
  cmd_odd = commander_ids[:, 1]
  mut_ids_flat = mutation_ids.reshape(-1)
  mut_table_flat = mutation_table.reshape(-1)
  return _embed(map_ids, cmd_even, cmd_odd, mut_ids_flat, ai_ids,
                map_table, commander_table, mut_table_flat, ai_table)
